# baseline (speedup 1.0000x reference)
import functools

import jax
import jax.numpy as jnp
from jax import lax
from jax.experimental import pallas as pl
from jax.experimental.pallas import tpu as pltpu
from jax.experimental.pallas import tpu_sc as plsc

BLANK = 0
DEPTH = 1000
T_DIM = 50
B_DIM = 1024
TC_T = 30
SC_T = T_DIM - TC_T
NUM_SLOTS = 5
CHUNK = 40
N_CHUNKS = DEPTH // CHUNK
N_UNITS = SC_T * N_CHUNKS
N_VREGS = B_DIM // 16
MAX_PAIRS = (N_UNITS // 32 + 2) // 2 + 1


def _tc_body(idx_ref, out_ref, scratch, sems):
    i = pl.program_id(0)
    for k in range(NUM_SLOTS):
        @pl.when(i > 0)
        def _wait_prev():
            pltpu.make_async_copy(
                scratch.at[k],
                out_ref.at[(i - 1) * NUM_SLOTS + k],
                sems.at[k],
            ).wait()

        row = idx_ref[k]
        shifted = jnp.where(row == BLANK, -1, row)
        iota = lax.broadcasted_iota(jnp.int32, (DEPTH, idx_ref.shape[2]), 0)
        scratch[k] = (shifted == iota).astype(jnp.float32)

        pltpu.make_async_copy(
            scratch.at[k],
            out_ref.at[i * NUM_SLOTS + k],
            sems.at[k],
        ).start()

    @pl.when(i == pl.num_programs(0) - 1)
    def _drain():
        for k in range(NUM_SLOTS):
            pltpu.make_async_copy(
                scratch.at[k],
                out_ref.at[i * NUM_SLOTS + k],
                sems.at[k],
            ).wait()


def _sc_onehot(idx_hbm, out_hbm, bufs, ids, sems):
    info = plsc.get_sparse_core_info()
    nc = info.num_cores
    nw = nc * info.num_subcores
    wid = lax.axis_index("s") * nc + lax.axis_index("c")

    base_units = N_UNITS // nw
    n_extra = N_UNITS - base_units * nw
    lo = base_units * wid + jnp.minimum(wid, n_extra)
    cnt = base_units + jnp.where(wid < n_extra, 1, 0)

    lane = lax.broadcasted_iota(jnp.int32, (16,), 0)
    ones16 = jnp.full((16,), 1.0, dtype=jnp.float32)
    zeros16 = jnp.zeros((16,), dtype=jnp.float32)

    for b in range(2):
        def _zero_row(r, _):
            for c in range(N_VREGS):
                bufs[b, r, pl.ds(c * 16, 16)] = zeros16
            return 0
        lax.fori_loop(0, CHUNK, _zero_row, 0)

    def _scatter(b, c0, vals):
        def _step(v, _):
            vec = ids[b, pl.ds(v * 16, 16)]
            m = (vec > 0) & (vec >= c0) & (vec < c0 + CHUNK)
            plsc.store_scatter(
                bufs.at[b], [vec - c0, v * 16 + lane], vals, mask=m)
            return 0
        lax.fori_loop(0, N_VREGS, _step, 0)

    def _pair(p, _):
        for b in range(2):
            kk = 2 * p + b
            u = lo + kk

            @pl.when(kk < cnt)
            def _do_unit():
                t = u // N_CHUNKS
                c0 = (u % N_CHUNKS) * CHUNK

                @pl.when(p > 0)
                def _recycle():
                    u_prev = u - 2
                    t_prev = u_prev // N_CHUNKS
                    c0_prev = (u_prev % N_CHUNKS) * CHUNK
                    pltpu.make_async_copy(
                        bufs.at[b],
                        out_hbm.at[t_prev, pl.ds(c0_prev, CHUNK)],
                        sems.at[b],
                    ).wait()
                    _scatter(b, c0_prev, zeros16)

                pltpu.sync_copy(idx_hbm.at[t], ids.at[b])
                _scatter(b, c0, ones16)
                pltpu.make_async_copy(
                    bufs.at[b],
                    out_hbm.at[t, pl.ds(c0, CHUNK)],
                    sems.at[b],
                ).start()
        return 0

    lax.fori_loop(0, MAX_PAIRS, _pair, 0)

    for b in range(2):
        kl = cnt - 1 - ((cnt - 1 - b) % 2)
        u = lo + kl
        t = u // N_CHUNKS
        c0 = (u % N_CHUNKS) * CHUNK
        pltpu.make_async_copy(
            bufs.at[b],
            out_hbm.at[t, pl.ds(c0, CHUNK)],
            sems.at[b],
        ).wait()


def kernel(outputs, outputs_length):
    b, t = outputs.shape
    idx_t = outputs.astype(jnp.int32).T

    tc_part = pl.pallas_call(
        _tc_body,
        grid=(TC_T // NUM_SLOTS,),
        in_specs=[pl.BlockSpec((NUM_SLOTS, 1, b), lambda i: (i, 0, 0))],
        out_specs=pl.BlockSpec(memory_space=pl.ANY),
        out_shape=jax.ShapeDtypeStruct((TC_T, DEPTH, b), jnp.float32),
        scratch_shapes=[
            pltpu.VMEM((NUM_SLOTS, DEPTH, b), jnp.float32),
            pltpu.SemaphoreType.DMA((NUM_SLOTS,)),
        ],
    )(idx_t[:TC_T].reshape(TC_T, 1, b))

    mesh = plsc.VectorSubcoreMesh(core_axis_name="c", subcore_axis_name="s")
    sc_part = functools.partial(
        pl.kernel,
        mesh=mesh,
        compiler_params=pltpu.CompilerParams(needs_layout_passes=False),
        out_type=jax.ShapeDtypeStruct((SC_T, DEPTH, B_DIM), jnp.float32),
        scratch_types=[
            pltpu.VMEM((2, CHUNK, B_DIM), jnp.float32),
            pltpu.VMEM((2, B_DIM), jnp.int32),
            pltpu.SemaphoreType.DMA((2,)),
        ],
    )(_sc_onehot)(idx_t[TC_T:])

    one_hot_t = jnp.concatenate([tc_part, sc_part], axis=0)
    return (jnp.transpose(one_hot_t, (2, 0, 1)), outputs_length)


# speedup vs baseline: 3.2068x; 3.2068x over previous
import jax
import jax.numpy as jnp
from jax import lax
from jax.experimental import pallas as pl
from jax.experimental.pallas import tpu as pltpu

BLANK = 0
DEPTH = 1000
NUM_SLOTS = 5


def _onehot_body(idx_ref, out_ref, scratch, sems):
    i = pl.program_id(0)
    for k in range(NUM_SLOTS):
        @pl.when(i > 0)
        def _wait_prev():
            pltpu.make_async_copy(
                scratch.at[k],
                out_ref.at[(i - 1) * NUM_SLOTS + k],
                sems.at[k],
            ).wait()

        row = idx_ref[k]
        shifted = jnp.where(row == BLANK, -1, row)
        iota = lax.broadcasted_iota(
            jnp.int32, (DEPTH, idx_ref.shape[2]), 0)
        scratch[k] = (shifted == iota).astype(jnp.float32)

        pltpu.make_async_copy(
            scratch.at[k],
            out_ref.at[i * NUM_SLOTS + k],
            sems.at[k],
        ).start()

    @pl.when(i == pl.num_programs(0) - 1)
    def _drain():
        for k in range(NUM_SLOTS):
            pltpu.make_async_copy(
                scratch.at[k],
                out_ref.at[i * NUM_SLOTS + k],
                sems.at[k],
            ).wait()


def kernel(outputs, outputs_length):
    b, t = outputs.shape
    idx3 = outputs.astype(jnp.int32).T.reshape(t, 1, b)
    one_hot_t = pl.pallas_call(
        _onehot_body,
        grid=(t // NUM_SLOTS,),
        in_specs=[pl.BlockSpec((NUM_SLOTS, 1, b), lambda i: (i, 0, 0))],
        out_specs=pl.BlockSpec(memory_space=pl.ANY),
        out_shape=jax.ShapeDtypeStruct((t, DEPTH, b), jnp.float32),
        scratch_shapes=[
            pltpu.VMEM((NUM_SLOTS, DEPTH, b), jnp.float32),
            pltpu.SemaphoreType.DMA((NUM_SLOTS,)),
        ],
    )(idx3)
    return (jnp.transpose(one_hot_t, (2, 0, 1)), outputs_length)


# speedup vs baseline: 3.2784x; 1.0223x over previous
import jax
import jax.numpy as jnp
from jax import lax
from jax.experimental import pallas as pl
from jax.experimental.pallas import tpu as pltpu

BLANK = 0
DEPTH = 1000
NUM_SLOTS = 5


def _onehot_body(idx_ref, out_ref, scratch, sems):
    i = pl.program_id(0)
    for k in range(NUM_SLOTS):
        @pl.when(i > 0)
        def _wait_prev():
            pltpu.make_async_copy(
                scratch.at[k],
                out_ref.at[(i - 1) * NUM_SLOTS + k],
                sems.at[k],
            ).wait()

        row = idx_ref[pl.ds(i * NUM_SLOTS + k, 1), :]
        shifted = jnp.where(row == BLANK, -1, row)
        iota = lax.broadcasted_iota(
            jnp.int32, (DEPTH, idx_ref.shape[1]), 0)
        scratch[k] = (shifted == iota).astype(jnp.float32)

        pltpu.make_async_copy(
            scratch.at[k],
            out_ref.at[i * NUM_SLOTS + k],
            sems.at[k],
        ).start()

    @pl.when(i == pl.num_programs(0) - 1)
    def _drain():
        for k in range(NUM_SLOTS):
            pltpu.make_async_copy(
                scratch.at[k],
                out_ref.at[i * NUM_SLOTS + k],
                sems.at[k],
            ).wait()


def kernel(outputs, outputs_length):
    b, t = outputs.shape
    idx_t = outputs.astype(jnp.int32).T
    one_hot_t = pl.pallas_call(
        _onehot_body,
        grid=(t // NUM_SLOTS,),
        in_specs=[pl.BlockSpec((t, b), lambda i: (0, 0))],
        out_specs=pl.BlockSpec(memory_space=pl.ANY),
        out_shape=jax.ShapeDtypeStruct((t, DEPTH, b), jnp.float32),
        scratch_shapes=[
            pltpu.VMEM((NUM_SLOTS, DEPTH, b), jnp.float32),
            pltpu.SemaphoreType.DMA((NUM_SLOTS,)),
        ],
    )(idx_t)
    return (jnp.transpose(one_hot_t, (2, 0, 1)), outputs_length)
